# baseline (device time: 42862 ns/iter reference)
import jax
import jax.numpy as jnp
from jax import lax
from jax.experimental import pallas as pl
from jax.experimental.pallas import tpu as pltpu

N_DEV = 8
SQ = 256
SKV_LOCAL = 4096
HQ = 8
DH = 128
DM = 1024
BLK = 64
NBLK = SKV_LOCAL // BLK
GCAP = 22
GROWS = GCAP * BLK
SCALE = 0.08838834764831843
LOG2E = 1.4426950408889634
NEG = -1e30
ROUNDS = 3
NCHUNK = 4

BF = jnp.bfloat16
F32 = jnp.float32

CLASS_ORDER = (0, 2, 1)
CLASS_CHUNKS = {0: ((0, 0), (3, BLK)), 2: ((1, 0),), 1: ((2, 0),)}
CHUNK_ORDER = (0, 3, 1, 2)


def kernel(x, Wq, K_ext, V_ext, Wo):

    def body(
        x_ref,
        wq_ref,
        k_ref,
        v_ref,
        wo_ref,
        out_ref,
        kg,
        vg,
        ke,
        ve,
        gsems,
        usend,
        urecv,
        lsend,
        lrecv,
        usend_sems,
        urecv_sems,
        lsend_sems,
        lrecv_sems,
    ):
        my = lax.axis_index("i")

        copies = {0: [], 1: [], 2: []}
        for r in CLASS_ORDER:
            b0 = (r - my) % 3
            b0r = b0 * BLK
            for j in range(GCAP):
                off = (
                    jnp.minimum(b0r + 3 * j * BLK, (NBLK - 1) * BLK)
                    if j == GCAP - 1
                    else b0r + 3 * j * BLK
                )
                for src, dst in ((k_ref, kg), (v_ref, vg)):
                    for h in range(HQ):
                        c = pltpu.make_async_copy(
                            src.at[0, pl.ds(off, BLK), h, :],
                            dst.at[r, pl.ds(j * BLK, BLK), pl.ds(h * DH, DH)],
                            gsems.at[r],
                        )
                        c.start()
                        copies[r].append(c)
        for src, dst in ((k_ref, ke), (v_ref, ve)):
            for slot, (cls, blocks) in enumerate(((2, (0, 1)), (1, (0, 2)))):
                for pos, blk in enumerate(blocks):
                    for h in range(HQ):
                        c = pltpu.make_async_copy(
                            src.at[0, pl.ds(blk * BLK, BLK), h, :],
                            dst.at[slot, pl.ds(pos * BLK, BLK), pl.ds(h * DH, DH)],
                            gsems.at[cls],
                        )
                        c.start()
                        copies[cls].append(c)

        q = jnp.dot(
            x_ref[0].astype(BF),
            wq_ref[...].astype(BF),
            preferred_element_type=F32,
        )
        qv = (q * (SCALE * LOG2E)).astype(BF)

        col = lax.broadcasted_iota(jnp.int32, (1, GROWS), 1)
        gbias = {}
        for r in range(3):
            valid = jnp.where(r == my % 3, GROWS, GROWS - BLK)
            gbias[r] = jnp.where(col < valid, 0.0, NEG).astype(BF)
        ebias = jnp.where(my == 0, 0.0, NEG).astype(BF)

        q_by_class = {
            0: jnp.concatenate([qv[0:BLK], qv[3 * BLK : 4 * BLK]], axis=0),
            1: qv[2 * BLK : 3 * BLK],
            2: qv[BLK : 2 * BLK],
        }
        extras_slot = {1: 1, 2: 0}

        u_chunk = {}
        l_chunk = {}
        rd_u = {}
        rd_l = {}

        def start_round(c, r):
            partner = my ^ (1 << r)
            usend[c, r, :, :] = u_chunk[c].astype(BF)
            lsend[c, r, :, :] = l_chunk[c]
            rd_u[(c, r)] = pltpu.make_async_remote_copy(
                src_ref=usend.at[c, r],
                dst_ref=urecv.at[c, r],
                send_sem=usend_sems.at[c, r],
                recv_sem=urecv_sems.at[c, r],
                device_id=(partner,),
                device_id_type=pl.DeviceIdType.MESH,
            )
            rd_l[(c, r)] = pltpu.make_async_remote_copy(
                src_ref=lsend.at[c, r],
                dst_ref=lrecv.at[c, r],
                send_sem=lsend_sems.at[c, r],
                recv_sem=lrecv_sems.at[c, r],
                device_id=(partner,),
                device_id_type=pl.DeviceIdType.MESH,
            )
            rd_u[(c, r)].start()
            rd_l[(c, r)].start()

        barrier = pltpu.get_barrier_semaphore()
        for r in range(ROUNDS):
            pl.semaphore_signal(
                barrier,
                inc=1,
                device_id=(my ^ (1 << r),),
                device_id_type=pl.DeviceIdType.MESH,
            )
        pl.semaphore_wait(barrier, ROUNDS)

        for cls in CLASS_ORDER:
            for cpy in copies[cls]:
                cpy.wait()
            qr = q_by_class[cls]
            u_heads = []
            l_heads = []
            for h in range(HQ):
                sl = slice(h * DH, (h + 1) * DH)
                s = lax.dot_general(
                    qr[:, sl],
                    kg[cls, :, sl].astype(BF),
                    (((1,), (1,)), ((), ())),
                    preferred_element_type=F32,
                )
                w = jnp.exp2(s.astype(BF) + gbias[cls])
                u = jnp.dot(
                    w,
                    vg[cls, :, sl].astype(BF),
                    preferred_element_type=F32,
                )
                l = jnp.sum(w, axis=1, keepdims=True, dtype=F32)
                if cls in extras_slot:
                    slot = extras_slot[cls]
                    se = lax.dot_general(
                        qr[:, sl],
                        ke[slot, :, sl].astype(BF),
                        (((1,), (1,)), ((), ())),
                        preferred_element_type=F32,
                    )
                    we = jnp.exp2(se.astype(BF) + ebias)
                    u = u + jnp.dot(
                        we,
                        ve[slot, :, sl].astype(BF),
                        preferred_element_type=F32,
                    )
                    l = l + jnp.sum(we, axis=1, keepdims=True, dtype=F32)
                u_heads.append(u)
                l_heads.append(l)
            for chunk, row0 in CLASS_CHUNKS[cls]:
                u_chunk[chunk] = jnp.concatenate(
                    [uh[row0 : row0 + BLK] for uh in u_heads], axis=1
                )
                l_chunk[chunk] = jnp.concatenate(
                    [lh[row0 : row0 + BLK] for lh in l_heads], axis=1
                )
                start_round(chunk, 0)

        wo_bf = wo_ref[...].astype(BF)
        for r in range(ROUNDS):
            for c in CHUNK_ORDER:
                rd_u[(c, r)].wait()
                rd_l[(c, r)].wait()
                u_chunk[c] = u_chunk[c] + urecv[c, r, :, :].astype(F32)
                l_chunk[c] = l_chunk[c] + lrecv[c, r, :, :]
                if r < ROUNDS - 1:
                    start_round(c, r + 1)
                else:
                    u = u_chunk[c]
                    l = l_chunk[c]
                    ctx = jnp.concatenate(
                        [
                            (u[:, h * DH : (h + 1) * DH] / l[:, h : h + 1]).astype(BF)
                            for h in range(HQ)
                        ],
                        axis=1,
                    )
                    out_ref[0, pl.ds(c * BLK, BLK), :] = jnp.dot(
                        ctx, wo_bf, preferred_element_type=F32
                    )

    out = pl.pallas_call(
        body,
        out_shape=jax.ShapeDtypeStruct((1, SQ, DM), F32),
        in_specs=[
            pl.BlockSpec(memory_space=pltpu.VMEM),
            pl.BlockSpec(memory_space=pltpu.VMEM),
            pl.BlockSpec(memory_space=pl.ANY),
            pl.BlockSpec(memory_space=pl.ANY),
            pl.BlockSpec(memory_space=pltpu.VMEM),
        ],
        out_specs=pl.BlockSpec(memory_space=pltpu.VMEM),
        scratch_shapes=[
            pltpu.VMEM((3, GROWS, DM), F32),
            pltpu.VMEM((3, GROWS, DM), F32),
            pltpu.VMEM((2, 2 * BLK, DM), F32),
            pltpu.VMEM((2, 2 * BLK, DM), F32),
            pltpu.SemaphoreType.DMA((3,)),
            pltpu.VMEM((NCHUNK, ROUNDS, BLK, DM), BF),
            pltpu.VMEM((NCHUNK, ROUNDS, BLK, DM), BF),
            pltpu.VMEM((NCHUNK, ROUNDS, BLK, HQ), F32),
            pltpu.VMEM((NCHUNK, ROUNDS, BLK, HQ), F32),
            pltpu.SemaphoreType.DMA((NCHUNK, ROUNDS)),
            pltpu.SemaphoreType.DMA((NCHUNK, ROUNDS)),
            pltpu.SemaphoreType.DMA((NCHUNK, ROUNDS)),
            pltpu.SemaphoreType.DMA((NCHUNK, ROUNDS)),
        ],
        compiler_params=pltpu.CompilerParams(
            collective_id=0, vmem_limit_bytes=100 * 1024 * 1024
        ),
    )(x, Wq, K_ext, V_ext, Wo)
    return out


# device time: 42437 ns/iter; 1.0100x vs baseline; 1.0100x over previous
import jax
import jax.numpy as jnp
from jax import lax
from jax.experimental import pallas as pl
from jax.experimental.pallas import tpu as pltpu

N_DEV = 8
SQ = 256
SKV_LOCAL = 4096
HQ = 8
DH = 128
DM = 1024
BLK = 64
NBLK = SKV_LOCAL // BLK
GCAP = 22
GROWS = GCAP * BLK
SCALE = 0.08838834764831843
LOG2E = 1.4426950408889634
NEG = -1e30
ROUNDS = 3
NCHUNK = 4

BF = jnp.bfloat16
F32 = jnp.float32

CLASS_ORDER = (0, 2, 1)
CLASS_CHUNKS = {0: ((0, 0), (3, BLK)), 2: ((1, 0),), 1: ((2, 0),)}
CHUNK_ORDER = (0, 3, 1, 2)


def kernel(x, Wq, K_ext, V_ext, Wo):

    def body(
        x_ref,
        wq_ref,
        k_ref,
        v_ref,
        wo_ref,
        out_ref,
        kg,
        vg,
        ke,
        ve,
        gsems,
        usend,
        urecv,
        lsend,
        lrecv,
        usend_sems,
        urecv_sems,
        lsend_sems,
        lrecv_sems,
    ):
        my = lax.axis_index("i")

        copies = {0: [], 1: [], 2: []}
        for r in CLASS_ORDER:
            b0 = (r - my) % 3
            b0r = b0 * BLK
            for j in range(GCAP):
                off = (
                    jnp.minimum(b0r + 3 * j * BLK, (NBLK - 1) * BLK)
                    if j == GCAP - 1
                    else b0r + 3 * j * BLK
                )
                for src, dst in ((k_ref, kg), (v_ref, vg)):
                    for h in range(HQ):
                        c = pltpu.make_async_copy(
                            src.at[0, pl.ds(off, BLK), h, :],
                            dst.at[r, pl.ds(j * BLK, BLK), pl.ds(h * DH, DH)],
                            gsems.at[r],
                        )
                        c.start()
                        copies[r].append(c)
        for src, dst in ((k_ref, ke), (v_ref, ve)):
            for slot, (cls, blocks) in enumerate(((2, (0, 1)), (1, (0, 2)))):
                for pos, blk in enumerate(blocks):
                    for h in range(HQ):
                        c = pltpu.make_async_copy(
                            src.at[0, pl.ds(blk * BLK, BLK), h, :],
                            dst.at[slot, pl.ds(pos * BLK, BLK), pl.ds(h * DH, DH)],
                            gsems.at[cls],
                        )
                        c.start()
                        copies[cls].append(c)

        q = jnp.dot(
            x_ref[0].astype(BF),
            wq_ref[...].astype(BF),
            preferred_element_type=F32,
        )
        qv = (q * (SCALE * LOG2E)).astype(BF)

        col = lax.broadcasted_iota(jnp.int32, (1, GROWS), 1)
        gbias = {}
        for r in range(3):
            valid = jnp.where(r == my % 3, GROWS, GROWS - BLK)
            gbias[r] = jnp.where(col < valid, 0.0, NEG).astype(BF)
        ebias = jnp.where(my == 0, 0.0, NEG).astype(BF)

        q_by_class = {
            0: jnp.concatenate([qv[0:BLK], qv[3 * BLK : 4 * BLK]], axis=0),
            1: qv[2 * BLK : 3 * BLK],
            2: qv[BLK : 2 * BLK],
        }
        extras_slot = {1: 1, 2: 0}

        u_chunk = {}
        l_chunk = {}
        rd_u = {}
        rd_l = {}

        def start_round_u(c, r):
            partner = my ^ (1 << r)
            usend[c, r, :, :] = u_chunk[c].astype(BF)
            rd_u[(c, r)] = pltpu.make_async_remote_copy(
                src_ref=usend.at[c, r],
                dst_ref=urecv.at[c, r],
                send_sem=usend_sems.at[c, r],
                recv_sem=urecv_sems.at[c, r],
                device_id=(partner,),
                device_id_type=pl.DeviceIdType.MESH,
            )
            rd_u[(c, r)].start()

        def start_round_l(c, r):
            partner = my ^ (1 << r)
            lsend[c, r, :, :] = l_chunk[c]
            rd_l[(c, r)] = pltpu.make_async_remote_copy(
                src_ref=lsend.at[c, r],
                dst_ref=lrecv.at[c, r],
                send_sem=lsend_sems.at[c, r],
                recv_sem=lrecv_sems.at[c, r],
                device_id=(partner,),
                device_id_type=pl.DeviceIdType.MESH,
            )
            rd_l[(c, r)].start()

        def start_round(c, r):
            start_round_u(c, r)
            start_round_l(c, r)

        barrier = pltpu.get_barrier_semaphore()
        for r in range(ROUNDS):
            pl.semaphore_signal(
                barrier,
                inc=1,
                device_id=(my ^ (1 << r),),
                device_id_type=pl.DeviceIdType.MESH,
            )
        pl.semaphore_wait(barrier, ROUNDS)

        for cls in CLASS_ORDER:
            for cpy in copies[cls]:
                cpy.wait()
            qr = q_by_class[cls]
            u_heads = []
            l_heads = []
            for h in range(HQ):
                sl = slice(h * DH, (h + 1) * DH)
                s = lax.dot_general(
                    qr[:, sl],
                    kg[cls, :, sl].astype(BF),
                    (((1,), (1,)), ((), ())),
                    preferred_element_type=F32,
                )
                w = jnp.exp2(s.astype(BF) + gbias[cls])
                u = jnp.dot(
                    w,
                    vg[cls, :, sl].astype(BF),
                    preferred_element_type=F32,
                )
                l = jnp.sum(w, axis=1, keepdims=True, dtype=F32)
                if cls in extras_slot:
                    slot = extras_slot[cls]
                    se = lax.dot_general(
                        qr[:, sl],
                        ke[slot, :, sl].astype(BF),
                        (((1,), (1,)), ((), ())),
                        preferred_element_type=F32,
                    )
                    we = jnp.exp2(se.astype(BF) + ebias)
                    u = u + jnp.dot(
                        we,
                        ve[slot, :, sl].astype(BF),
                        preferred_element_type=F32,
                    )
                    l = l + jnp.sum(we, axis=1, keepdims=True, dtype=F32)
                u_heads.append(u)
                l_heads.append(l)
            for chunk, row0 in CLASS_CHUNKS[cls]:
                u_chunk[chunk] = jnp.concatenate(
                    [uh[row0 : row0 + BLK] for uh in u_heads], axis=1
                )
                l_chunk[chunk] = jnp.concatenate(
                    [lh[row0 : row0 + BLK] for lh in l_heads], axis=1
                )
                start_round(chunk, 0)

        wo_bf = wo_ref[...].astype(BF)
        for r in range(ROUNDS):
            for c in CHUNK_ORDER:
                rd_u[(c, r)].wait()
                u_chunk[c] = u_chunk[c] + urecv[c, r, :, :].astype(F32)
                if r < ROUNDS - 1:
                    start_round_u(c, r + 1)
                rd_l[(c, r)].wait()
                l_chunk[c] = l_chunk[c] + lrecv[c, r, :, :]
                if r < ROUNDS - 1:
                    start_round_l(c, r + 1)
                else:
                    u = u_chunk[c]
                    l = l_chunk[c]
                    ctx = jnp.concatenate(
                        [
                            (u[:, h * DH : (h + 1) * DH] / l[:, h : h + 1]).astype(BF)
                            for h in range(HQ)
                        ],
                        axis=1,
                    )
                    out_ref[0, pl.ds(c * BLK, BLK), :] = jnp.dot(
                        ctx, wo_bf, preferred_element_type=F32
                    )

    out = pl.pallas_call(
        body,
        out_shape=jax.ShapeDtypeStruct((1, SQ, DM), F32),
        in_specs=[
            pl.BlockSpec(memory_space=pltpu.VMEM),
            pl.BlockSpec(memory_space=pltpu.VMEM),
            pl.BlockSpec(memory_space=pl.ANY),
            pl.BlockSpec(memory_space=pl.ANY),
            pl.BlockSpec(memory_space=pltpu.VMEM),
        ],
        out_specs=pl.BlockSpec(memory_space=pltpu.VMEM),
        scratch_shapes=[
            pltpu.VMEM((3, GROWS, DM), F32),
            pltpu.VMEM((3, GROWS, DM), F32),
            pltpu.VMEM((2, 2 * BLK, DM), F32),
            pltpu.VMEM((2, 2 * BLK, DM), F32),
            pltpu.SemaphoreType.DMA((3,)),
            pltpu.VMEM((NCHUNK, ROUNDS, BLK, DM), BF),
            pltpu.VMEM((NCHUNK, ROUNDS, BLK, DM), BF),
            pltpu.VMEM((NCHUNK, ROUNDS, BLK, HQ), F32),
            pltpu.VMEM((NCHUNK, ROUNDS, BLK, HQ), F32),
            pltpu.SemaphoreType.DMA((NCHUNK, ROUNDS)),
            pltpu.SemaphoreType.DMA((NCHUNK, ROUNDS)),
            pltpu.SemaphoreType.DMA((NCHUNK, ROUNDS)),
            pltpu.SemaphoreType.DMA((NCHUNK, ROUNDS)),
        ],
        compiler_params=pltpu.CompilerParams(
            collective_id=0, vmem_limit_bytes=100 * 1024 * 1024
        ),
    )(x, Wq, K_ext, V_ext, Wo)
    return out
